# baseline (speedup 1.0000x reference)
import functools

import jax
import jax.numpy as jnp
from jax import lax
from jax.experimental import pallas as pl
from jax.experimental.pallas import tpu as pltpu
from jax.experimental.pallas import tpu_sc as plsc

N_ENTITY = 1000000
N_RELATION = 1000
DIM = 64
GAMMA = 12.0
EMB_RANGE = (GAMMA + 2.0) / DIM
PI = 3.141592653589793
BATCH = 4096
PHASE_K = PI / EMB_RANGE

NC, NS, L = 2, 16, 16
NW = NC * NS
BPW = BATCH // NW
NCHUNK = DIM // L


def _horner(coeffs, t):
    acc = jnp.full((L,), coeffs[-1], jnp.float32)
    for c in coeffs[-2::-1]:
        acc = acc * t + jnp.float32(c)
    return acc


_SQRT12_C = (0.26855847239494324, 1.1340605020523071, -0.6584334969520569,
             0.3633367717266083, -0.13294294476509094, 0.027977269142866135,
             -0.0025564369279891253)


def _modulus16(re, im):
    a = jnp.abs(re)
    b = jnp.abs(im)
    hi = jnp.maximum(a, b)
    lo = jnp.minimum(a, b)
    ratio = lo / (hi + jnp.float32(1e-30))
    return hi * _horner(_SQRT12_C, jnp.float32(1.0) + ratio * ratio)


def _trig_body(rel_ref, cs_ref):
    ph = rel_ref[...] * jnp.float32(PHASE_K)
    cs_ref[...] = jnp.concatenate([jnp.cos(ph), jnp.sin(ph)], axis=1)


_trig_table = pl.pallas_call(
    _trig_body,
    out_shape=jax.ShapeDtypeStruct((N_RELATION, 2 * DIM), jnp.float32),
)

_mesh = plsc.VectorSubcoreMesh(core_axis_name="c", subcore_axis_name="s")


@functools.partial(
    pl.kernel,
    out_type=jax.ShapeDtypeStruct((BATCH,), jnp.float32),
    mesh=_mesh,
    compiler_params=pltpu.CompilerParams(needs_layout_passes=False,
                                         use_tc_tiling_on_sc=False),
    scratch_types=[
        pltpu.VMEM((BPW,), jnp.int32),
        pltpu.VMEM((BPW,), jnp.int32),
        pltpu.VMEM((BPW,), jnp.int32),
        pltpu.VMEM((BPW, 2 * DIM), jnp.float32),
        pltpu.VMEM((BPW, 2 * DIM), jnp.float32),
        pltpu.VMEM((BPW, 2 * DIM), jnp.float32),
        pltpu.VMEM((BPW, L), jnp.float32),
        pltpu.VMEM((BPW,), jnp.float32),
        pltpu.SemaphoreType.DMA,
        pltpu.SemaphoreType.DMA,
        pltpu.SemaphoreType.DMA,
    ],
)
def _rotate_body(h_hbm, r_hbm, t_hbm, ent_hbm, cs_hbm, out_hbm,
                 hidx, ridx, tidx, head_v, tail_v, cs_v, part_v, out_v,
                 sem_h, sem_t, sem_r):
    wid = lax.axis_index("s") * NC + lax.axis_index("c")
    base = wid * BPW

    pltpu.sync_copy(h_hbm.at[pl.ds(base, BPW)], hidx)
    pltpu.sync_copy(t_hbm.at[pl.ds(base, BPW)], tidx)
    pltpu.sync_copy(r_hbm.at[pl.ds(base, BPW)], ridx)

    cp_h = pltpu.async_copy(ent_hbm.at[hidx], head_v, sem_h)
    cp_t = pltpu.async_copy(ent_hbm.at[tidx], tail_v, sem_t)
    cp_r = pltpu.async_copy(cs_hbm.at[ridx], cs_v, sem_r)
    cp_r.wait()
    cp_h.wait()
    cp_t.wait()

    def item(i, carry):
        acc = jnp.zeros((L,), jnp.float32)
        for j in range(NCHUNK):
            lo = j * L
            cos_r = cs_v[i, pl.ds(lo, L)]
            sin_r = cs_v[i, pl.ds(DIM + lo, L)]
            re_t = tail_v[i, pl.ds(lo, L)]
            im_t = tail_v[i, pl.ds(DIM + lo, L)]
            re_h = head_v[i, pl.ds(lo, L)]
            im_h = head_v[i, pl.ds(DIM + lo, L)]
            re_s = cos_r * re_t + sin_r * im_t - re_h
            im_s = cos_r * im_t - sin_r * re_t - im_h
            acc = acc + _modulus16(re_s, im_s)
        part_v[i, pl.ds(0, L)] = acc
        return carry

    lax.fori_loop(0, BPW, item, 0, unroll=2)

    iota = lax.iota(jnp.int32, L)
    for g in range(BPW // L):
        items = iota + jnp.int32(g * L)
        tot = jnp.zeros((L,), jnp.float32)
        for d in range(L):
            tot = tot + plsc.load_gather(
                part_v, [items, jnp.full((L,), d, jnp.int32)])
        out_v[pl.ds(g * L, L)] = -tot

    pltpu.sync_copy(out_v, out_hbm.at[pl.ds(base, BPW)])


def kernel(h, r, t, entity_embedding, relation_embedding):
    cs_table = _trig_table(relation_embedding)
    flat = _rotate_body(h.astype(jnp.int32), r.astype(jnp.int32),
                        t.astype(jnp.int32), entity_embedding, cs_table)
    return flat.reshape(BATCH, 1)
